# baseline (speedup 1.0000x reference)
import functools

import jax
import jax.numpy as jnp
from jax import lax
from jax.experimental import pallas as pl
from jax.experimental.pallas import tpu as pltpu
from jax.experimental.pallas import tpu_sc as plsc

B = 16
N_NODE = 1000
EMB = 32
MAX_NEI = 16
N_EDGE = MAX_NEI * (N_NODE - 1)
FSS = 51
SLEFT = N_NODE * EMB
NP = 1024
RB = 256

_F_LAYERS = [("gc1", 64, 128, True), ("gc2", 128, 64, True),
             ("gc3", 64, 64, True), ("gc4", 64, 64, False)]
_R_LAYERS = [("r_gc1", 64, 64, True), ("r_gc2", 64, 64, True),
             ("r_gc3", 64, 64, True), ("r_gc4", 64, 64, False)]
_LAYERS = _F_LAYERS + _R_LAYERS
_LN_NAMES = ["ln1", "ln2", "ln3", "r_ln1", "r_ln2", "r_ln3"]


def _pad2(w, r, c):
    return jnp.zeros((r, c), jnp.float32).at[: w.shape[0], : w.shape[1]].set(w)


def _pad1(v, c):
    return jnp.zeros((1, c), jnp.float32).at[0, : v.shape[0]].set(v)


def _rowmask(row0):
    return (lax.broadcasted_iota(jnp.int32, (RB, 1), 0) + row0
            < N_NODE).astype(jnp.float32)


def _gat_stack_body(sa_ref, c_ref, vecs_ref, w0, w1, w2, w3, w4, w5, w6, w7,
                    f_ref, r_ref, x_s, ha_s):
    wrefs = [w0, w1, w2, w3, w4, w5, w6, w7]
    nblk = NP // RB

    def attention_layer(li, src_ref, din, dout, is_last, is_r_last):
        wv = wrefs[li]
        x_full = src_ref[:, :din]
        h_aug = jnp.dot(x_full, wv[:, :], preferred_element_type=jnp.float32)
        ha_s[:, : dout + 8] = h_aug
        ha_s[:, dout : dout + 1] = jnp.ones((NP, 1), jnp.float32)
        a_src = vecs_ref[3 * li : 3 * li + 1, :dout]
        b_row = vecs_ref[3 * li + 2 : 3 * li + 3, :dout]
        hh = ha_s[:, :dout]
        asrc = lax.dot_general(a_src, hh, (((1,), (1,)), ((), ())),
                               preferred_element_type=jnp.float32)
        asrc = jnp.minimum(asrc, 43.3)
        z = jnp.zeros((1, dout), jnp.float32)
        s1, s2 = z, z
        for rb in range(nblk):
            row0 = rb * RB
            adc = jnp.minimum(ha_s[row0 : row0 + RB, dout + 1 : dout + 2],
                              43.3)
            acc = jnp.zeros((RB, dout + 1), jnp.float32)
            for shi in range(NP // 128):
                cc = c_ref[0, shi, row0 : row0 + RB, :]
                e = adc + asrc[:, shi * 128 : shi * 128 + 128]
                e = jnp.maximum(e, 0.2 * e)
                w = cc * jnp.exp2(e)
                acc = acc + jnp.dot(
                    w, ha_s[shi * 128 : shi * 128 + 128, : dout + 1],
                    preferred_element_type=jnp.float32)
            den = acc[:, dout : dout + 1]
            out = acc[:, :dout] * (1.0 / (den + 1e-16)) + b_row
            if is_last and not is_r_last:
                f_ref[0, pl.ds(row0, RB), :] = out
                continue
            if rb == nblk - 1:
                out = out * _rowmask(row0)
            if is_r_last:
                s1 = s1 + jnp.sum(out, axis=0, keepdims=True)
                continue
            x_s[pl.ds(row0, RB), :dout] = out
            s1 = s1 + jnp.sum(out, axis=0, keepdims=True)
            s2 = s2 + jnp.sum(out * out, axis=0, keepdims=True)
        return s1, s2

    def apply_ln(ln_i, dout, s1, s2):
        scale = vecs_ref[24 + 2 * ln_i : 25 + 2 * ln_i, :dout]
        offset = vecs_ref[25 + 2 * ln_i : 26 + 2 * ln_i, :dout]
        mean = s1 * (1.0 / N_NODE)
        var = s2 * (1.0 / N_NODE) - mean * mean
        mul = scale * lax.rsqrt(var + 1e-5)
        for rb in range(nblk):
            row0 = rb * RB
            xc = x_s[pl.ds(row0, RB), :dout]
            y = (xc - mean) * mul + offset
            y = jnp.maximum(y, 0.0)
            if rb == nblk - 1:
                y = y * _rowmask(row0)
            x_s[pl.ds(row0, RB), :dout] = y

    def tower(layers, li0, ln_i0, is_r):
        for k, (_, din, dout, has_ln) in enumerate(layers):
            li = li0 + k
            src = sa_ref.at[0] if k == 0 else x_s
            is_last = not has_ln
            s1, s2 = attention_layer(li, src, din, dout, is_last,
                                     is_last and is_r)
            if has_ln:
                apply_ln(ln_i0 + k, dout, s1, s2)
            if is_last and is_r:
                r_ref[0, :, :] = jnp.broadcast_to(s1, (8, 64))

    tower(_F_LAYERS, 0, 0, False)
    tower(_R_LAYERS, 4, 3, True)


@functools.partial(jax.jit, static_argnames=("interpret",))
def _gat_stack(sa_p, c_p, vecs, ws, interpret=False):
    ng = sa_p.shape[0]
    wspecs = [pl.BlockSpec(w.shape, lambda g: (0, 0)) for w in ws]
    f_out, r_out = pl.pallas_call(
        _gat_stack_body,
        grid=(ng,),
        in_specs=[
            pl.BlockSpec((1, NP, 64), lambda g: (g, 0, 0)),
            pl.BlockSpec((1, NP // 128, NP, 128), lambda g: (g, 0, 0, 0)),
            pl.BlockSpec(vecs.shape, lambda g: (0, 0)),
        ] + wspecs,
        out_specs=[
            pl.BlockSpec((1, NP, 64), lambda g: (g, 0, 0)),
            pl.BlockSpec((1, 8, 64), lambda g: (g, 0, 0)),
        ],
        out_shape=[
            jax.ShapeDtypeStruct((ng, NP, 64), jnp.float32),
            jax.ShapeDtypeStruct((ng, 8, 64), jnp.float32),
        ],
        scratch_shapes=[
            pltpu.VMEM((NP, 128), jnp.float32),
            pltpu.VMEM((NP, 136), jnp.float32),
        ],
        interpret=interpret,
    )(sa_p, c_p, vecs, *ws)
    return f_out, r_out


def _build_counts_jnp(snd, rcv):
    idx = rcv * NP + snd
    c = jax.vmap(lambda ix: jnp.zeros((NP * NP,), jnp.float32).at[ix].add(1.0))(idx)
    return c.reshape(B, NP, NP)


NSUB = 16
NCORE = 2
EPT = 1024
GPC = B // NCORE
GSLICE = NP * NP // NSUB


def _counts_body(gpc, idx_hbm, vals_hbm, zer_hbm, c_hbm, idx_v, val_v, zbuf,
                 cbuf_sh):
    cid = lax.axis_index("c")
    sid = lax.axis_index("s")
    pltpu.sync_copy(zer_hbm, zbuf)
    pltpu.sync_copy(vals_hbm, val_v)
    base = sid * GSLICE
    for k in range(GSLICE // 1024):
        pltpu.sync_copy(zbuf, cbuf_sh.at[pl.ds(base + k * 1024, 1024)])
    plsc.subcore_barrier()
    for i in range(gpc):
        g = cid * gpc + i
        pltpu.sync_copy(idx_hbm.at[g, sid], idx_v)
        for j in range(EPT // 128):
            pltpu.sync_copy(val_v.at[0, j], cbuf_sh.at[idx_v.at[j]], add=True)
        plsc.subcore_barrier()
        pltpu.sync_copy(cbuf_sh.at[pl.ds(base, GSLICE)],
                        c_hbm.at[pl.ds(g * (NP * NP) + base, GSLICE)])
        plsc.subcore_barrier()
        if i < gpc - 1:
            for j in range(EPT // 128):
                pltpu.sync_copy(val_v.at[1, j], cbuf_sh.at[idx_v.at[j]], add=True)


@jax.jit
def _build_counts_sc(idx_p, vals, zer):
    ng = idx_p.shape[0]
    mesh = plsc.VectorSubcoreMesh(core_axis_name="c", subcore_axis_name="s")
    return pl.kernel(
        functools.partial(_counts_body, ng // NCORE),
        jax.ShapeDtypeStruct((ng * NP * NP,), jnp.float32),
        mesh=mesh,
        scratch_types=[
            pltpu.VMEM((EPT // 128, 128), jnp.int32),
            pltpu.VMEM((2, EPT // 128, 128), jnp.float32),
            pltpu.VMEM((1024,), jnp.float32),
            pltpu.VMEM_SHARED((NP * NP,), jnp.float32),
        ],
    )(idx_p, vals, zer)


def kernel(ns, a, params):
    nodes = ns[:, :SLEFT].reshape(B, N_NODE, EMB)
    snd = ns[:, SLEFT : SLEFT + N_EDGE].astype(jnp.int32)
    rcv = ns[:, SLEFT + N_EDGE : SLEFT + 2 * N_EDGE].astype(jnp.int32)
    onehot = (jnp.arange(N_NODE)[None, :] == a[:, None]).astype(jnp.float32)

    sa_p = jnp.zeros((B, NP, 64), jnp.float32)
    sa_p = sa_p.at[:, :N_NODE, :EMB].set(nodes)
    sa_p = sa_p.at[:, :N_NODE, EMB].set(onehot)

    idx = ((snd >> 7) * (NP * 128) + rcv * 128 + (snd & 127)).reshape(
        B, NSUB, N_EDGE // NSUB)
    idx_p = jnp.pad(idx, ((0, 0), (0, 0), (0, EPT - N_EDGE // NSUB)))
    idx_p = idx_p.reshape(B, NSUB, EPT // 128, 128)
    vpat = (jnp.arange(EPT) < N_EDGE // NSUB).astype(jnp.float32)
    vals = jnp.stack([vpat, -vpat]).reshape(2, EPT // 128, 128)
    zer = jnp.zeros((1024,), jnp.float32)
    hb = B // 4
    c_a = _build_counts_sc(idx_p[:hb], vals, zer).reshape(hb, NP // 128, NP, 128)
    c_b = _build_counts_sc(idx_p[hb:], vals, zer).reshape(B - hb, NP // 128,
                                                          NP, 128)

    ws, vec_rows = [], []
    for (nm, din, dout, _) in _LAYERS:
        p = params[nm]
        w_aug = jnp.zeros((din, dout + 8), jnp.float32)
        w_aug = w_aug.at[: p["W"].shape[0], : p["W"].shape[1]].set(p["W"])
        w_aug = w_aug.at[: p["W"].shape[0], dout + 1].set(
            (p["W"] @ p["a_dst"]) * 1.4426950408889634)
        ws.append(w_aug)
        vec_rows += [_pad1(p["a_src"] * 1.4426950408889634, 128),
                     _pad1(p["a_dst"], 128), _pad1(p["b"], 128)]
    for nm in _LN_NAMES:
        p = params[nm]
        vec_rows += [_pad1(p["scale"], 128), _pad1(p["offset"], 128)]
    vecs = jnp.concatenate(vec_rows + [jnp.zeros((4, 128), jnp.float32)], axis=0)

    f_a, r_a = _gat_stack(sa_p[:hb], c_a, vecs, tuple(ws))
    f_b, r_b = _gat_stack(sa_p[hb:], c_b, vecs, tuple(ws))
    f_out = jnp.concatenate([f_a, f_b], axis=0)
    r_out = jnp.concatenate([r_a, r_b], axis=0)
    f = f_out[:, :N_NODE, :EMB].reshape(B, N_NODE * EMB)
    r = r_out[:, 0, :FSS]
    ns_out = jnp.concatenate([f, ns[:, SLEFT:]], axis=1)
    return (r, ns_out)


# speedup vs baseline: 1.1243x; 1.1243x over previous
import functools

import jax
import jax.numpy as jnp
from jax import lax
from jax.experimental import pallas as pl
from jax.experimental.pallas import tpu as pltpu
from jax.experimental.pallas import tpu_sc as plsc

B = 16
N_NODE = 1000
EMB = 32
MAX_NEI = 16
N_EDGE = MAX_NEI * (N_NODE - 1)
FSS = 51
SLEFT = N_NODE * EMB
NP = 1024
RB = 256

_F_LAYERS = [("gc1", 64, 128, True), ("gc2", 128, 64, True),
             ("gc3", 64, 64, True), ("gc4", 64, 64, False)]
_R_LAYERS = [("r_gc1", 64, 64, True), ("r_gc2", 64, 64, True),
             ("r_gc3", 64, 64, True), ("r_gc4", 64, 64, False)]
_LAYERS = _F_LAYERS + _R_LAYERS
_LN_NAMES = ["ln1", "ln2", "ln3", "r_ln1", "r_ln2", "r_ln3"]


def _pad2(w, r, c):
    return jnp.zeros((r, c), jnp.float32).at[: w.shape[0], : w.shape[1]].set(w)


def _pad1(v, c):
    return jnp.zeros((1, c), jnp.float32).at[0, : v.shape[0]].set(v)


def _rowmask(row0):
    return (lax.broadcasted_iota(jnp.int32, (RB, 1), 0) + row0
            < N_NODE).astype(jnp.float32)


def _gat_stack_body(sa_ref, c_ref, vecs_ref, w0, w1, w2, w3, w4, w5, w6, w7,
                    f_ref, r_ref, x_f, x_r, ha_f, ha_r):
    wrefs = [w0, w1, w2, w3, w4, w5, w6, w7]
    nblk = NP // RB

    def prep(li, src_ref, din, dout, ha):
        h_aug = jnp.dot(src_ref[:, :din], wrefs[li][:, :],
                        preferred_element_type=jnp.float32)
        ha[:, : dout + 8] = h_aug
        ha[:, dout : dout + 1] = jnp.ones((NP, 1), jnp.float32)
        a_src = vecs_ref[3 * li : 3 * li + 1, :dout]
        asrc = lax.dot_general(a_src, ha[:, :dout], (((1,), (1,)), ((), ())),
                               preferred_element_type=jnp.float32)
        return jnp.minimum(asrc, 43.3)

    def apply_ln(ln_i, dout, s1, s2, xs):
        scale = vecs_ref[24 + 2 * ln_i : 25 + 2 * ln_i, :dout]
        offset = vecs_ref[25 + 2 * ln_i : 26 + 2 * ln_i, :dout]
        mean = s1 * (1.0 / N_NODE)
        var = s2 * (1.0 / N_NODE) - mean * mean
        mul = scale * lax.rsqrt(var + 1e-5)
        for rb in range(nblk):
            row0 = rb * RB
            y = (xs[pl.ds(row0, RB), :dout] - mean) * mul + offset
            y = jnp.maximum(y, 0.0)
            if rb == nblk - 1:
                y = y * _rowmask(row0)
            xs[pl.ds(row0, RB), :dout] = y

    def joint_layer(k):
        _, din_f, dout_f, has_ln = _F_LAYERS[k]
        _, din_r, dout_r, _ = _R_LAYERS[k]
        src_f = sa_ref.at[0] if k == 0 else x_f
        src_r = sa_ref.at[0] if k == 0 else x_r
        asrc_f = prep(k, src_f, din_f, dout_f, ha_f)
        asrc_r = prep(4 + k, src_r, din_r, dout_r, ha_r)
        b_f = vecs_ref[3 * k + 2 : 3 * k + 3, :dout_f]
        b_r = vecs_ref[3 * (4 + k) + 2 : 3 * (4 + k) + 3, :dout_r]
        z_f = jnp.zeros((1, dout_f), jnp.float32)
        z_r = jnp.zeros((1, dout_r), jnp.float32)
        s1f, s2f, s1r, s2r, rsum = z_f, z_f, z_r, z_r, z_r
        last = not has_ln
        for rb in range(nblk):
            row0 = rb * RB
            adc_f = jnp.minimum(
                ha_f[row0 : row0 + RB, dout_f + 1 : dout_f + 2], 43.3)
            adc_r = jnp.minimum(
                ha_r[row0 : row0 + RB, dout_r + 1 : dout_r + 2], 43.3)
            acc_f = jnp.zeros((RB, dout_f + 1), jnp.float32)
            acc_r = jnp.zeros((RB, dout_r + 1), jnp.float32)
            for shi in range(NP // 128):
                cc = c_ref[0, shi, row0 : row0 + RB, :]
                sl = slice(shi * 128, shi * 128 + 128)
                ef = adc_f + asrc_f[:, sl]
                ef = jnp.maximum(ef, 0.2 * ef)
                wf = cc * jnp.exp2(ef)
                acc_f = acc_f + jnp.dot(wf, ha_f[sl, : dout_f + 1],
                                        preferred_element_type=jnp.float32)
                er = adc_r + asrc_r[:, sl]
                er = jnp.maximum(er, 0.2 * er)
                wr = cc * jnp.exp2(er)
                acc_r = acc_r + jnp.dot(wr, ha_r[sl, : dout_r + 1],
                                        preferred_element_type=jnp.float32)
            out_f = (acc_f[:, :dout_f]
                     * (1.0 / (acc_f[:, dout_f : dout_f + 1] + 1e-16)) + b_f)
            out_r = (acc_r[:, :dout_r]
                     * (1.0 / (acc_r[:, dout_r : dout_r + 1] + 1e-16)) + b_r)
            if last:
                f_ref[0, pl.ds(row0, RB), :] = out_f
                if rb == nblk - 1:
                    out_r = out_r * _rowmask(row0)
                rsum = rsum + jnp.sum(out_r, axis=0, keepdims=True)
            else:
                if rb == nblk - 1:
                    m = _rowmask(row0)
                    out_f = out_f * m
                    out_r = out_r * m
                x_f[pl.ds(row0, RB), :dout_f] = out_f
                x_r[pl.ds(row0, RB), :dout_r] = out_r
                s1f = s1f + jnp.sum(out_f, axis=0, keepdims=True)
                s2f = s2f + jnp.sum(out_f * out_f, axis=0, keepdims=True)
                s1r = s1r + jnp.sum(out_r, axis=0, keepdims=True)
                s2r = s2r + jnp.sum(out_r * out_r, axis=0, keepdims=True)
        if has_ln:
            apply_ln(k, dout_f, s1f, s2f, x_f)
            apply_ln(3 + k, dout_r, s1r, s2r, x_r)
        else:
            r_ref[0, :, :] = jnp.broadcast_to(rsum, (8, 64))

    for k in range(4):
        joint_layer(k)


@functools.partial(jax.jit, static_argnames=("interpret",))
def _gat_stack(sa_p, c_p, vecs, ws, interpret=False):
    ng = sa_p.shape[0]
    wspecs = [pl.BlockSpec(w.shape, lambda g: (0, 0)) for w in ws]
    f_out, r_out = pl.pallas_call(
        _gat_stack_body,
        grid=(ng,),
        in_specs=[
            pl.BlockSpec((1, NP, 64), lambda g: (g, 0, 0)),
            pl.BlockSpec((1, NP // 128, NP, 128), lambda g: (g, 0, 0, 0)),
            pl.BlockSpec(vecs.shape, lambda g: (0, 0)),
        ] + wspecs,
        out_specs=[
            pl.BlockSpec((1, NP, 64), lambda g: (g, 0, 0)),
            pl.BlockSpec((1, 8, 64), lambda g: (g, 0, 0)),
        ],
        out_shape=[
            jax.ShapeDtypeStruct((ng, NP, 64), jnp.float32),
            jax.ShapeDtypeStruct((ng, 8, 64), jnp.float32),
        ],
        scratch_shapes=[
            pltpu.VMEM((NP, 128), jnp.float32),
            pltpu.VMEM((NP, 64), jnp.float32),
            pltpu.VMEM((NP, 136), jnp.float32),
            pltpu.VMEM((NP, 72), jnp.float32),
        ],
        interpret=interpret,
    )(sa_p, c_p, vecs, *ws)
    return f_out, r_out


def _build_counts_jnp(snd, rcv):
    idx = rcv * NP + snd
    c = jax.vmap(lambda ix: jnp.zeros((NP * NP,), jnp.float32).at[ix].add(1.0))(idx)
    return c.reshape(B, NP, NP)


NSUB = 16
NCORE = 2
EPT = 1024
GPC = B // NCORE
GSLICE = NP * NP // NSUB


def _counts_body(gpc, idx_hbm, vals_hbm, zer_hbm, c_hbm, idx_v, val_v, zbuf,
                 cbuf_sh):
    cid = lax.axis_index("c")
    sid = lax.axis_index("s")
    pltpu.sync_copy(zer_hbm, zbuf)
    pltpu.sync_copy(vals_hbm, val_v)
    base = sid * GSLICE
    for k in range(GSLICE // 1024):
        pltpu.sync_copy(zbuf, cbuf_sh.at[pl.ds(base + k * 1024, 1024)])
    plsc.subcore_barrier()
    for i in range(gpc):
        g = cid * gpc + i
        pltpu.sync_copy(idx_hbm.at[g, sid], idx_v)
        for j in range(EPT // 128):
            pltpu.sync_copy(val_v.at[0, j], cbuf_sh.at[idx_v.at[j]], add=True)
        plsc.subcore_barrier()
        pltpu.sync_copy(cbuf_sh.at[pl.ds(base, GSLICE)],
                        c_hbm.at[pl.ds(g * (NP * NP) + base, GSLICE)])
        plsc.subcore_barrier()
        if i < gpc - 1:
            for j in range(EPT // 128):
                pltpu.sync_copy(val_v.at[1, j], cbuf_sh.at[idx_v.at[j]], add=True)


@jax.jit
def _build_counts_sc(idx_p, vals, zer):
    ng = idx_p.shape[0]
    mesh = plsc.VectorSubcoreMesh(core_axis_name="c", subcore_axis_name="s")
    return pl.kernel(
        functools.partial(_counts_body, ng // NCORE),
        jax.ShapeDtypeStruct((ng * NP * NP,), jnp.float32),
        mesh=mesh,
        scratch_types=[
            pltpu.VMEM((EPT // 128, 128), jnp.int32),
            pltpu.VMEM((2, EPT // 128, 128), jnp.float32),
            pltpu.VMEM((1024,), jnp.float32),
            pltpu.VMEM_SHARED((NP * NP,), jnp.float32),
        ],
    )(idx_p, vals, zer)


def kernel(ns, a, params):
    nodes = ns[:, :SLEFT].reshape(B, N_NODE, EMB)
    snd = ns[:, SLEFT : SLEFT + N_EDGE].astype(jnp.int32)
    rcv = ns[:, SLEFT + N_EDGE : SLEFT + 2 * N_EDGE].astype(jnp.int32)
    onehot = (jnp.arange(N_NODE)[None, :] == a[:, None]).astype(jnp.float32)

    sa_p = jnp.zeros((B, NP, 64), jnp.float32)
    sa_p = sa_p.at[:, :N_NODE, :EMB].set(nodes)
    sa_p = sa_p.at[:, :N_NODE, EMB].set(onehot)

    idx = ((snd >> 7) * (NP * 128) + rcv * 128 + (snd & 127)).reshape(
        B, NSUB, N_EDGE // NSUB)
    idx_p = jnp.pad(idx, ((0, 0), (0, 0), (0, EPT - N_EDGE // NSUB)))
    idx_p = idx_p.reshape(B, NSUB, EPT // 128, 128)
    vpat = (jnp.arange(EPT) < N_EDGE // NSUB).astype(jnp.float32)
    vals = jnp.stack([vpat, -vpat]).reshape(2, EPT // 128, 128)
    zer = jnp.zeros((1024,), jnp.float32)
    c_p = _build_counts_sc(idx_p, vals, zer).reshape(B, NP // 128, NP, 128)

    ws, vec_rows = [], []
    for (nm, din, dout, _) in _LAYERS:
        p = params[nm]
        w_aug = jnp.zeros((din, dout + 8), jnp.float32)
        w_aug = w_aug.at[: p["W"].shape[0], : p["W"].shape[1]].set(p["W"])
        w_aug = w_aug.at[: p["W"].shape[0], dout + 1].set(
            (p["W"] @ p["a_dst"]) * 1.4426950408889634)
        ws.append(w_aug)
        vec_rows += [_pad1(p["a_src"] * 1.4426950408889634, 128),
                     _pad1(p["a_dst"], 128), _pad1(p["b"], 128)]
    for nm in _LN_NAMES:
        p = params[nm]
        vec_rows += [_pad1(p["scale"], 128), _pad1(p["offset"], 128)]
    vecs = jnp.concatenate(vec_rows + [jnp.zeros((4, 128), jnp.float32)], axis=0)

    f_out, r_out = _gat_stack(sa_p, c_p, vecs, tuple(ws))
    f = f_out[:, :N_NODE, :EMB].reshape(B, N_NODE * EMB)
    r = r_out[:, 0, :FSS]
    ns_out = jnp.concatenate([f, ns[:, SLEFT:]], axis=1)
    return (r, ns_out)


# speedup vs baseline: 1.1268x; 1.0022x over previous
import functools

import jax
import jax.numpy as jnp
from jax import lax
from jax.experimental import pallas as pl
from jax.experimental.pallas import tpu as pltpu
from jax.experimental.pallas import tpu_sc as plsc

B = 16
N_NODE = 1000
EMB = 32
MAX_NEI = 16
N_EDGE = MAX_NEI * (N_NODE - 1)
FSS = 51
SLEFT = N_NODE * EMB
NP = 1024
RB = 256

_F_LAYERS = [("gc1", 64, 128, True), ("gc2", 128, 64, True),
             ("gc3", 64, 64, True), ("gc4", 64, 64, False)]
_R_LAYERS = [("r_gc1", 64, 64, True), ("r_gc2", 64, 64, True),
             ("r_gc3", 64, 64, True), ("r_gc4", 64, 64, False)]
_LAYERS = _F_LAYERS + _R_LAYERS
_LN_NAMES = ["ln1", "ln2", "ln3", "r_ln1", "r_ln2", "r_ln3"]


def _pad2(w, r, c):
    return jnp.zeros((r, c), jnp.float32).at[: w.shape[0], : w.shape[1]].set(w)


def _pad1(v, c):
    return jnp.zeros((1, c), jnp.float32).at[0, : v.shape[0]].set(v)


def _rowmask(row0):
    return (lax.broadcasted_iota(jnp.int32, (RB, 1), 0) + row0
            < N_NODE).astype(jnp.float32)


def _gat_stack_body(sa_ref, c_ref, vecs_ref, w0, w1, w2, w3, w4, w5, w6, w7,
                    f_ref, r_ref, x_f, x_r, ha_f, ha_r):
    wrefs = [w0, w1, w2, w3, w4, w5, w6, w7]
    nblk = NP // RB

    def prep(li, src_ref, din, dout, ha):
        h_aug = jnp.dot(src_ref[:, :din], wrefs[li][:, :],
                        preferred_element_type=jnp.float32)
        ha[:, : dout + 8] = h_aug
        ha[:, dout : dout + 1] = jnp.ones((NP, 1), jnp.float32)
        a_src = vecs_ref[3 * li : 3 * li + 1, :dout]
        asrc = lax.dot_general(a_src, ha[:, :dout], (((1,), (1,)), ((), ())),
                               preferred_element_type=jnp.float32)
        return jnp.minimum(asrc, 43.3)

    def apply_ln(ln_i, dout, s1, s2, xs):
        scale = vecs_ref[24 + 2 * ln_i : 25 + 2 * ln_i, :dout]
        offset = vecs_ref[25 + 2 * ln_i : 26 + 2 * ln_i, :dout]
        mean = s1 * (1.0 / N_NODE)
        var = s2 * (1.0 / N_NODE) - mean * mean
        mul = scale * lax.rsqrt(var + 1e-5)
        for rb in range(nblk):
            row0 = rb * RB
            y = (xs[pl.ds(row0, RB), :dout] - mean) * mul + offset
            y = jnp.maximum(y, 0.0)
            if rb == nblk - 1:
                y = y * _rowmask(row0)
            xs[pl.ds(row0, RB), :dout] = y

    def joint_layer(k):
        _, din_f, dout_f, has_ln = _F_LAYERS[k]
        _, din_r, dout_r, _ = _R_LAYERS[k]
        src_f = sa_ref.at[0] if k == 0 else x_f
        src_r = sa_ref.at[0] if k == 0 else x_r
        asrc_f = prep(k, src_f, din_f, dout_f, ha_f)
        asrc_r = prep(4 + k, src_r, din_r, dout_r, ha_r)
        b_f = vecs_ref[3 * k + 2 : 3 * k + 3, :dout_f]
        b_r = vecs_ref[3 * (4 + k) + 2 : 3 * (4 + k) + 3, :dout_r]
        z_f = jnp.zeros((1, dout_f), jnp.float32)
        z_r = jnp.zeros((1, dout_r), jnp.float32)
        s1f, s2f, s1r, s2r, rsum = z_f, z_f, z_r, z_r, z_r
        last = not has_ln
        for rb in range(nblk):
            row0 = rb * RB
            adc_f = jnp.minimum(
                ha_f[row0 : row0 + RB, dout_f + 1 : dout_f + 2], 43.3)
            adc_r = jnp.minimum(
                ha_r[row0 : row0 + RB, dout_r + 1 : dout_r + 2], 43.3)
            acc_f = jnp.zeros((RB, dout_f + 1), jnp.float32)
            acc_r = jnp.zeros((RB, dout_r + 1), jnp.float32)
            for shi in range(NP // 128):
                cc = c_ref[0, shi, row0 : row0 + RB, :]
                sl = slice(shi * 128, shi * 128 + 128)
                ef = adc_f + asrc_f[:, sl]
                ef = jnp.maximum(ef, 0.2 * ef)
                wf = cc * jnp.exp2(ef)
                acc_f = acc_f + jnp.dot(wf, ha_f[sl, : dout_f + 1],
                                        preferred_element_type=jnp.float32)
                er = adc_r + asrc_r[:, sl]
                er = jnp.maximum(er, 0.2 * er)
                wr = cc * jnp.exp2(er)
                acc_r = acc_r + jnp.dot(wr, ha_r[sl, : dout_r + 1],
                                        preferred_element_type=jnp.float32)
            out_f = (acc_f[:, :dout_f]
                     * (1.0 / (acc_f[:, dout_f : dout_f + 1] + 1e-16)) + b_f)
            out_r = (acc_r[:, :dout_r]
                     * (1.0 / (acc_r[:, dout_r : dout_r + 1] + 1e-16)) + b_r)
            if last:
                f_ref[0, pl.ds(row0, RB), :] = out_f
                if rb == nblk - 1:
                    out_r = out_r * _rowmask(row0)
                rsum = rsum + jnp.sum(out_r, axis=0, keepdims=True)
            else:
                if rb == nblk - 1:
                    m = _rowmask(row0)
                    out_f = out_f * m
                    out_r = out_r * m
                x_f[pl.ds(row0, RB), :dout_f] = out_f
                x_r[pl.ds(row0, RB), :dout_r] = out_r
                s1f = s1f + jnp.sum(out_f, axis=0, keepdims=True)
                s2f = s2f + jnp.sum(out_f * out_f, axis=0, keepdims=True)
                s1r = s1r + jnp.sum(out_r, axis=0, keepdims=True)
                s2r = s2r + jnp.sum(out_r * out_r, axis=0, keepdims=True)
        if has_ln:
            apply_ln(k, dout_f, s1f, s2f, x_f)
            apply_ln(3 + k, dout_r, s1r, s2r, x_r)
        else:
            r_ref[0, :, :] = jnp.broadcast_to(rsum, (8, 64))

    for k in range(4):
        joint_layer(k)


@functools.partial(jax.jit, static_argnames=("interpret",))
def _gat_stack(sa_p, c_p, vecs, ws, interpret=False):
    ng = sa_p.shape[0]
    wspecs = [pl.BlockSpec(w.shape, lambda g: (0, 0)) for w in ws]
    f_out, r_out = pl.pallas_call(
        _gat_stack_body,
        grid=(ng,),
        in_specs=[
            pl.BlockSpec((1, NP, 64), lambda g: (g, 0, 0)),
            pl.BlockSpec((1, NP // 128, NP, 128), lambda g: (g, 0, 0, 0)),
            pl.BlockSpec(vecs.shape, lambda g: (0, 0)),
        ] + wspecs,
        out_specs=[
            pl.BlockSpec((1, NP, 64), lambda g: (g, 0, 0)),
            pl.BlockSpec((1, 8, 64), lambda g: (g, 0, 0)),
        ],
        out_shape=[
            jax.ShapeDtypeStruct((ng, NP, 64), jnp.float32),
            jax.ShapeDtypeStruct((ng, 8, 64), jnp.float32),
        ],
        scratch_shapes=[
            pltpu.VMEM((NP, 128), jnp.float32),
            pltpu.VMEM((NP, 64), jnp.float32),
            pltpu.VMEM((NP, 136), jnp.float32),
            pltpu.VMEM((NP, 72), jnp.float32),
        ],
        interpret=interpret,
    )(sa_p, c_p, vecs, *ws)
    return f_out, r_out


def _build_counts_jnp(snd, rcv):
    idx = rcv * NP + snd
    c = jax.vmap(lambda ix: jnp.zeros((NP * NP,), jnp.float32).at[ix].add(1.0))(idx)
    return c.reshape(B, NP, NP)


NSUB = 16
NCORE = 2
EPT = 1024
GPC = B // NCORE
GSLICE = NP * NP // NSUB


def _counts_body(gpc, idx_hbm, vals_hbm, zer_hbm, c_hbm, idx_v, val_v, zbuf,
                 cbuf_sh, sem_p, sem_m):
    cid = lax.axis_index("c")
    sid = lax.axis_index("s")
    pltpu.sync_copy(zer_hbm, zbuf)
    pltpu.sync_copy(vals_hbm, val_v)
    base = sid * GSLICE
    for k in range(GSLICE // 8192):
        pltpu.sync_copy(zbuf, cbuf_sh.at[pl.ds(base + k * 8192, 8192)])
    plsc.subcore_barrier()
    minus_cps = []
    for i in range(gpc):
        g = cid * gpc + i
        ib = i % 2
        pltpu.sync_copy(idx_hbm.at[g, sid], idx_v.at[ib])
        plus_cps = [
            pltpu.async_copy(val_v.at[0, j], cbuf_sh.at[idx_v.at[ib, j]],
                             sem_p, add=True)
            for j in range(EPT // 128)
        ]
        for cp in minus_cps:
            cp.wait()
        for cp in plus_cps:
            cp.wait()
        plsc.subcore_barrier()
        pltpu.sync_copy(cbuf_sh.at[pl.ds(base, GSLICE)],
                        c_hbm.at[pl.ds(g * (NP * NP) + base, GSLICE)])
        plsc.subcore_barrier()
        if i < gpc - 1:
            minus_cps = [
                pltpu.async_copy(val_v.at[1, j], cbuf_sh.at[idx_v.at[ib, j]],
                                 sem_m, add=True)
                for j in range(EPT // 128)
            ]


@jax.jit
def _build_counts_sc(idx_p, vals, zer):
    ng = idx_p.shape[0]
    mesh = plsc.VectorSubcoreMesh(core_axis_name="c", subcore_axis_name="s")
    return pl.kernel(
        functools.partial(_counts_body, ng // NCORE),
        jax.ShapeDtypeStruct((ng * NP * NP,), jnp.float32),
        mesh=mesh,
        scratch_types=[
            pltpu.VMEM((2, EPT // 128, 128), jnp.int32),
            pltpu.VMEM((2, EPT // 128, 128), jnp.float32),
            pltpu.VMEM((8192,), jnp.float32),
            pltpu.VMEM_SHARED((NP * NP,), jnp.float32),
            pltpu.SemaphoreType.DMA,
            pltpu.SemaphoreType.DMA,
        ],
    )(idx_p, vals, zer)


def kernel(ns, a, params):
    nodes = ns[:, :SLEFT].reshape(B, N_NODE, EMB)
    snd = ns[:, SLEFT : SLEFT + N_EDGE].astype(jnp.int32)
    rcv = ns[:, SLEFT + N_EDGE : SLEFT + 2 * N_EDGE].astype(jnp.int32)
    onehot = (jnp.arange(N_NODE)[None, :] == a[:, None]).astype(jnp.float32)

    sa_p = jnp.zeros((B, NP, 64), jnp.float32)
    sa_p = sa_p.at[:, :N_NODE, :EMB].set(nodes)
    sa_p = sa_p.at[:, :N_NODE, EMB].set(onehot)

    idx = ((snd >> 7) * (NP * 128) + rcv * 128 + (snd & 127)).reshape(
        B, NSUB, N_EDGE // NSUB)
    idx_p = jnp.pad(idx, ((0, 0), (0, 0), (0, EPT - N_EDGE // NSUB)))
    idx_p = idx_p.reshape(B, NSUB, EPT // 128, 128)
    vpat = (jnp.arange(EPT) < N_EDGE // NSUB).astype(jnp.float32)
    vals = jnp.stack([vpat, -vpat]).reshape(2, EPT // 128, 128)
    zer = jnp.zeros((8192,), jnp.float32)
    c_p = _build_counts_sc(idx_p, vals, zer).reshape(B, NP // 128, NP, 128)

    ws, vec_rows = [], []
    for (nm, din, dout, _) in _LAYERS:
        p = params[nm]
        w_aug = jnp.zeros((din, dout + 8), jnp.float32)
        w_aug = w_aug.at[: p["W"].shape[0], : p["W"].shape[1]].set(p["W"])
        w_aug = w_aug.at[: p["W"].shape[0], dout + 1].set(
            (p["W"] @ p["a_dst"]) * 1.4426950408889634)
        ws.append(w_aug)
        vec_rows += [_pad1(p["a_src"] * 1.4426950408889634, 128),
                     _pad1(p["a_dst"], 128), _pad1(p["b"], 128)]
    for nm in _LN_NAMES:
        p = params[nm]
        vec_rows += [_pad1(p["scale"], 128), _pad1(p["offset"], 128)]
    vecs = jnp.concatenate(vec_rows + [jnp.zeros((4, 128), jnp.float32)], axis=0)

    f_out, r_out = _gat_stack(sa_p, c_p, vecs, tuple(ws))
    f = f_out[:, :N_NODE, :EMB].reshape(B, N_NODE * EMB)
    r = r_out[:, 0, :FSS]
    ns_out = jnp.concatenate([f, ns[:, SLEFT:]], axis=1)
    return (r, ns_out)


# speedup vs baseline: 1.1366x; 1.0087x over previous
import functools

import jax
import jax.numpy as jnp
from jax import lax
from jax.experimental import pallas as pl
from jax.experimental.pallas import tpu as pltpu
from jax.experimental.pallas import tpu_sc as plsc

B = 16
N_NODE = 1000
EMB = 32
MAX_NEI = 16
N_EDGE = MAX_NEI * (N_NODE - 1)
FSS = 51
SLEFT = N_NODE * EMB
NP = 1024
RB = 128

_F_LAYERS = [("gc1", 64, 128, True), ("gc2", 128, 64, True),
             ("gc3", 64, 64, True), ("gc4", 64, 64, False)]
_R_LAYERS = [("r_gc1", 64, 64, True), ("r_gc2", 64, 64, True),
             ("r_gc3", 64, 64, True), ("r_gc4", 64, 64, False)]
_LAYERS = _F_LAYERS + _R_LAYERS
_LN_NAMES = ["ln1", "ln2", "ln3", "r_ln1", "r_ln2", "r_ln3"]


def _pad2(w, r, c):
    return jnp.zeros((r, c), jnp.float32).at[: w.shape[0], : w.shape[1]].set(w)


def _pad1(v, c):
    return jnp.zeros((1, c), jnp.float32).at[0, : v.shape[0]].set(v)


def _rowmask(row0):
    return (lax.broadcasted_iota(jnp.int32, (RB, 1), 0) + row0
            < N_NODE).astype(jnp.float32)


def _gat_stack_body(sa_ref, c_ref, vecs_ref, w0, w1, w2, w3, w4, w5, w6, w7,
                    f_ref, r_ref, x_f, x_r, ha_f, ha_r):
    wrefs = [w0, w1, w2, w3, w4, w5, w6, w7]
    nblk = NP // RB

    def prep(li, src_ref, din, dout, ha):
        h_aug = jnp.dot(src_ref[:, :din], wrefs[li][:, :],
                        preferred_element_type=jnp.float32)
        ha[:, : dout + 8] = h_aug
        ha[:, dout : dout + 1] = jnp.ones((NP, 1), jnp.float32)
        a_src = vecs_ref[3 * li : 3 * li + 1, :dout]
        asrc = lax.dot_general(a_src, ha[:, :dout], (((1,), (1,)), ((), ())),
                               preferred_element_type=jnp.float32)
        return jnp.minimum(asrc, 43.3)

    def apply_ln(ln_i, dout, s1, s2, xs):
        scale = vecs_ref[24 + 2 * ln_i : 25 + 2 * ln_i, :dout]
        offset = vecs_ref[25 + 2 * ln_i : 26 + 2 * ln_i, :dout]
        mean = s1 * (1.0 / N_NODE)
        var = s2 * (1.0 / N_NODE) - mean * mean
        mul = scale * lax.rsqrt(var + 1e-5)
        for rb in range(nblk):
            row0 = rb * RB
            y = (xs[pl.ds(row0, RB), :dout] - mean) * mul + offset
            y = jnp.maximum(y, 0.0)
            if rb == nblk - 1:
                y = y * _rowmask(row0)
            xs[pl.ds(row0, RB), :dout] = y

    def joint_layer(k):
        _, din_f, dout_f, has_ln = _F_LAYERS[k]
        _, din_r, dout_r, _ = _R_LAYERS[k]
        src_f = sa_ref.at[0] if k == 0 else x_f
        src_r = sa_ref.at[0] if k == 0 else x_r
        asrc_f = prep(k, src_f, din_f, dout_f, ha_f)
        asrc_r = prep(4 + k, src_r, din_r, dout_r, ha_r)
        b_f = vecs_ref[3 * k + 2 : 3 * k + 3, :dout_f]
        b_r = vecs_ref[3 * (4 + k) + 2 : 3 * (4 + k) + 3, :dout_r]
        z_f = jnp.zeros((1, dout_f), jnp.float32)
        z_r = jnp.zeros((1, dout_r), jnp.float32)
        s1f, s2f, s1r, s2r, rsum = z_f, z_f, z_r, z_r, z_r
        last = not has_ln
        for rb in range(nblk):
            row0 = rb * RB
            adc_f = jnp.minimum(
                ha_f[row0 : row0 + RB, dout_f + 1 : dout_f + 2], 43.3)
            adc_r = jnp.minimum(
                ha_r[row0 : row0 + RB, dout_r + 1 : dout_r + 2], 43.3)
            acc_f = jnp.zeros((RB, dout_f + 1), jnp.float32)
            acc_r = jnp.zeros((RB, dout_r + 1), jnp.float32)
            for shi in range(NP // 128):
                cc = c_ref[0, shi, row0 : row0 + RB, :]
                sl = slice(shi * 128, shi * 128 + 128)
                ef = adc_f + asrc_f[:, sl]
                ef = jnp.maximum(ef, 0.2 * ef)
                wf = cc * jnp.exp2(ef)
                acc_f = acc_f + jnp.dot(wf, ha_f[sl, : dout_f + 1],
                                        preferred_element_type=jnp.float32)
                er = adc_r + asrc_r[:, sl]
                er = jnp.maximum(er, 0.2 * er)
                wr = cc * jnp.exp2(er)
                acc_r = acc_r + jnp.dot(wr, ha_r[sl, : dout_r + 1],
                                        preferred_element_type=jnp.float32)
            out_f = (acc_f[:, :dout_f]
                     * (1.0 / (acc_f[:, dout_f : dout_f + 1] + 1e-16)) + b_f)
            out_r = (acc_r[:, :dout_r]
                     * (1.0 / (acc_r[:, dout_r : dout_r + 1] + 1e-16)) + b_r)
            if last:
                f_ref[0, pl.ds(row0, RB), :] = out_f
                if rb == nblk - 1:
                    out_r = out_r * _rowmask(row0)
                rsum = rsum + jnp.sum(out_r, axis=0, keepdims=True)
            else:
                if rb == nblk - 1:
                    m = _rowmask(row0)
                    out_f = out_f * m
                    out_r = out_r * m
                x_f[pl.ds(row0, RB), :dout_f] = out_f
                x_r[pl.ds(row0, RB), :dout_r] = out_r
                s1f = s1f + jnp.sum(out_f, axis=0, keepdims=True)
                s2f = s2f + jnp.sum(out_f * out_f, axis=0, keepdims=True)
                s1r = s1r + jnp.sum(out_r, axis=0, keepdims=True)
                s2r = s2r + jnp.sum(out_r * out_r, axis=0, keepdims=True)
        if has_ln:
            apply_ln(k, dout_f, s1f, s2f, x_f)
            apply_ln(3 + k, dout_r, s1r, s2r, x_r)
        else:
            r_ref[0, :, :] = jnp.broadcast_to(rsum, (8, 64))

    for k in range(4):
        joint_layer(k)


@functools.partial(jax.jit, static_argnames=("interpret",))
def _gat_stack(sa_p, c_p, vecs, ws, interpret=False):
    ng = sa_p.shape[0]
    wspecs = [pl.BlockSpec(w.shape, lambda g: (0, 0)) for w in ws]
    f_out, r_out = pl.pallas_call(
        _gat_stack_body,
        grid=(ng,),
        in_specs=[
            pl.BlockSpec((1, NP, 64), lambda g: (g, 0, 0)),
            pl.BlockSpec((1, NP // 128, NP, 128), lambda g: (g, 0, 0, 0)),
            pl.BlockSpec(vecs.shape, lambda g: (0, 0)),
        ] + wspecs,
        out_specs=[
            pl.BlockSpec((1, NP, 64), lambda g: (g, 0, 0)),
            pl.BlockSpec((1, 8, 64), lambda g: (g, 0, 0)),
        ],
        out_shape=[
            jax.ShapeDtypeStruct((ng, NP, 64), jnp.float32),
            jax.ShapeDtypeStruct((ng, 8, 64), jnp.float32),
        ],
        scratch_shapes=[
            pltpu.VMEM((NP, 128), jnp.float32),
            pltpu.VMEM((NP, 64), jnp.float32),
            pltpu.VMEM((NP, 136), jnp.float32),
            pltpu.VMEM((NP, 72), jnp.float32),
        ],
        interpret=interpret,
    )(sa_p, c_p, vecs, *ws)
    return f_out, r_out


def _build_counts_jnp(snd, rcv):
    idx = rcv * NP + snd
    c = jax.vmap(lambda ix: jnp.zeros((NP * NP,), jnp.float32).at[ix].add(1.0))(idx)
    return c.reshape(B, NP, NP)


NSUB = 16
NCORE = 2
EPT = 1024
GPC = B // NCORE
GSLICE = NP * NP // NSUB


def _counts_body(gpc, idx_hbm, vals_hbm, zer_hbm, c_hbm, idx_v, val_v, zbuf,
                 cbuf_sh, sem_p, sem_m):
    cid = lax.axis_index("c")
    sid = lax.axis_index("s")
    pltpu.sync_copy(zer_hbm, zbuf)
    pltpu.sync_copy(vals_hbm, val_v)
    base = sid * GSLICE
    for k in range(GSLICE // 8192):
        pltpu.sync_copy(zbuf, cbuf_sh.at[pl.ds(base + k * 8192, 8192)])
    plsc.subcore_barrier()
    minus_cps = []
    for i in range(gpc):
        g = cid * gpc + i
        ib = i % 2
        pltpu.sync_copy(idx_hbm.at[g, sid], idx_v.at[ib])
        plus_cps = [
            pltpu.async_copy(val_v.at[0, j], cbuf_sh.at[idx_v.at[ib, j]],
                             sem_p, add=True)
            for j in range(EPT // 128)
        ]
        for cp in minus_cps:
            cp.wait()
        for cp in plus_cps:
            cp.wait()
        plsc.subcore_barrier()
        pltpu.sync_copy(cbuf_sh.at[pl.ds(base, GSLICE)],
                        c_hbm.at[pl.ds(g * (NP * NP) + base, GSLICE)])
        plsc.subcore_barrier()
        if i < gpc - 1:
            minus_cps = [
                pltpu.async_copy(val_v.at[1, j], cbuf_sh.at[idx_v.at[ib, j]],
                                 sem_m, add=True)
                for j in range(EPT // 128)
            ]


@jax.jit
def _build_counts_sc(idx_p, vals, zer):
    ng = idx_p.shape[0]
    mesh = plsc.VectorSubcoreMesh(core_axis_name="c", subcore_axis_name="s")
    return pl.kernel(
        functools.partial(_counts_body, ng // NCORE),
        jax.ShapeDtypeStruct((ng * NP * NP,), jnp.float32),
        mesh=mesh,
        scratch_types=[
            pltpu.VMEM((2, EPT // 128, 128), jnp.int32),
            pltpu.VMEM((2, EPT // 128, 128), jnp.float32),
            pltpu.VMEM((8192,), jnp.float32),
            pltpu.VMEM_SHARED((NP * NP,), jnp.float32),
            pltpu.SemaphoreType.DMA,
            pltpu.SemaphoreType.DMA,
        ],
    )(idx_p, vals, zer)


def kernel(ns, a, params):
    nodes = ns[:, :SLEFT].reshape(B, N_NODE, EMB)
    snd = ns[:, SLEFT : SLEFT + N_EDGE].astype(jnp.int32)
    rcv = ns[:, SLEFT + N_EDGE : SLEFT + 2 * N_EDGE].astype(jnp.int32)
    onehot = (jnp.arange(N_NODE)[None, :] == a[:, None]).astype(jnp.float32)

    sa_p = jnp.zeros((B, NP, 64), jnp.float32)
    sa_p = sa_p.at[:, :N_NODE, :EMB].set(nodes)
    sa_p = sa_p.at[:, :N_NODE, EMB].set(onehot)

    idx = ((snd >> 7) * (NP * 128) + rcv * 128 + (snd & 127)).reshape(
        B, NSUB, N_EDGE // NSUB)
    idx_p = jnp.pad(idx, ((0, 0), (0, 0), (0, EPT - N_EDGE // NSUB)))
    idx_p = idx_p.reshape(B, NSUB, EPT // 128, 128)
    vpat = (jnp.arange(EPT) < N_EDGE // NSUB).astype(jnp.float32)
    vals = jnp.stack([vpat, -vpat]).reshape(2, EPT // 128, 128)
    zer = jnp.zeros((8192,), jnp.float32)
    c_p = _build_counts_sc(idx_p, vals, zer).reshape(B, NP // 128, NP, 128)

    ws, vec_rows = [], []
    for (nm, din, dout, _) in _LAYERS:
        p = params[nm]
        w_aug = jnp.zeros((din, dout + 8), jnp.float32)
        w_aug = w_aug.at[: p["W"].shape[0], : p["W"].shape[1]].set(p["W"])
        w_aug = w_aug.at[: p["W"].shape[0], dout + 1].set(
            (p["W"] @ p["a_dst"]) * 1.4426950408889634)
        ws.append(w_aug)
        vec_rows += [_pad1(p["a_src"] * 1.4426950408889634, 128),
                     _pad1(p["a_dst"], 128), _pad1(p["b"], 128)]
    for nm in _LN_NAMES:
        p = params[nm]
        vec_rows += [_pad1(p["scale"], 128), _pad1(p["offset"], 128)]
    vecs = jnp.concatenate(vec_rows + [jnp.zeros((4, 128), jnp.float32)], axis=0)

    f_out, r_out = _gat_stack(sa_p, c_p, vecs, tuple(ws))
    f = f_out[:, :N_NODE, :EMB].reshape(B, N_NODE * EMB)
    r = r_out[:, 0, :FSS]
    ns_out = jnp.concatenate([f, ns[:, SLEFT:]], axis=1)
    return (r, ns_out)
